# baseline (speedup 1.0000x reference)
import functools

import jax
import jax.numpy as jnp
from jax.experimental import pallas as pl
from jax.experimental.pallas import tpu as pltpu


def _se_body(x_ref, w1_ref, b1_ref, w2_ref, b2_ref, o_ref, *, inv_hw):
    x = x_ref[...]
    pooled = jnp.sum(x, axis=2, dtype=jnp.float32) * inv_hw
    h = jax.lax.dot_general(pooled, w1_ref[...], (((1,), (1,)), ((), ())),
                            preferred_element_type=jnp.float32) + b1_ref[...]
    h = jnp.maximum(h, 0.0)
    s = jax.lax.dot_general(h, w2_ref[...], (((1,), (1,)), ((), ())),
                            preferred_element_type=jnp.float32) + b2_ref[...]
    s = jax.nn.sigmoid(s)
    o_ref[...] = x * s.astype(x.dtype)[:, :, None]


def kernel(x, w1, b1, w2, b2):
    B, C, H, W = x.shape
    S = w1.shape[0]
    HW = H * W
    itemsize = jnp.dtype(x.dtype).itemsize

    x_flat = x.reshape(B, C, HW)
    b1r = b1.reshape(1, S)
    b2r = b2.reshape(1, C)

    per_image = C * HW * itemsize
    budget = 10 * 1024 * 1024
    Bt = 1
    for d in range(min(B, max(1, B // 8)), 0, -1):
        if B % d == 0 and d * per_image <= budget:
            Bt = d
            break
    grid = (B // Bt,)

    cost = pl.CostEstimate(
        flops=int(2 * B * C * HW + 4 * B * C * S),
        bytes_accessed=int(2 * B * C * HW * itemsize),
        transcendentals=int(B * C),
    )

    out_flat = pl.pallas_call(
        functools.partial(_se_body, inv_hw=1.0 / float(HW)),
        out_shape=jax.ShapeDtypeStruct((B, C, HW), x.dtype),
        grid=grid,
        in_specs=[
            pl.BlockSpec((Bt, C, HW), lambda b: (b, 0, 0)),
            pl.BlockSpec((S, C), lambda b: (0, 0)),
            pl.BlockSpec((1, S), lambda b: (0, 0)),
            pl.BlockSpec((C, S), lambda b: (0, 0)),
            pl.BlockSpec((1, C), lambda b: (0, 0)),
        ],
        out_specs=pl.BlockSpec((Bt, C, HW), lambda b: (b, 0, 0)),
        compiler_params=pltpu.CompilerParams(
            dimension_semantics=("parallel",),
            vmem_limit_bytes=48 * 1024 * 1024,
        ),
        cost_estimate=cost,
    )(x_flat, w1, b1r, w2, b2r)
    return out_flat.reshape(B, C, H, W)
